# Initial kernel scaffold; baseline (speedup 1.0000x reference)
import jax
import jax.numpy as jnp
from jax.experimental import pallas as pl


def kernel(nodesBatch, backwards_edgeBatch, problemTypeBatch, W1, b1, W2, b2, W3, b3):
    raise NotImplementedError("write your pallas kernel here")


# speedup vs baseline: 16.0505x; 16.0505x over previous
import functools

import jax
import jax.numpy as jnp
from jax import lax
from jax.experimental import pallas as pl
from jax.experimental.pallas import tpu as pltpu
from jax.experimental.pallas import tpu_sc as plsc

PASSES = 4
NUM_EDGE_SETS = 2
B, N, D, E = 4, 10000, 150, 320000
NP = 10240
DP = 160
EG = NUM_EDGE_SETS * E
NC, NS, L = 2, 16, 16
EPT = EG // NS
CH = 64
NCHUNK = EPT // CH
RPT = NP // NS
RC = 64
NRC = RPT // RC
GPC = B // NC


def _mp_kernel(nodes_hbm, srcg_hbm, dstl_hbm, partial_hbm, nscr_hbm,
               inc_shared, src_buf, dst_buf, rows, sbuf, gsem):
  c = lax.axis_index("c")
  t = lax.axis_index("s")

  for gi in range(GPC):
    g = c * GPC + gi
    for p in range(PASSES):
      nsrc = nodes_hbm if p == 0 else nscr_hbm

      lr00 = t * RPT
      pltpu.sync_copy(nsrc.at[pl.ds(g * NP + lr00, RPT)],
                      inc_shared.at[pl.ds(lr00, RPT)])
      plsc.subcore_barrier()

      ebase = g * EG + t * EPT

      @pl.loop(0, NCHUNK)
      def _chunk(i):
        eoff = pl.multiple_of(ebase + i * CH, 8)
        pltpu.sync_copy(srcg_hbm.at[pl.ds(eoff, CH)], src_buf)
        pltpu.sync_copy(dstl_hbm.at[pl.ds(eoff, CH)], dst_buf)
        pltpu.async_copy(nsrc.at[src_buf], rows, gsem).wait()
        pltpu.sync_copy(rows, inc_shared.at[dst_buf], add=True)

      plsc.subcore_barrier()

      if p < PASSES - 1:
        pltpu.sync_copy(inc_shared.at[pl.ds(lr00, RPT)],
                        nscr_hbm.at[pl.ds(g * NP + lr00, RPT)])
      else:
        for cc in range(DP // L):
          sbuf[pl.ds(cc * L, L)] = jnp.zeros((L,), jnp.float32)
        for k in range(NRC):
          lr0 = t * RPT + k * RC
          pltpu.sync_copy(inc_shared.at[pl.ds(lr0, RC)], rows)

          @pl.loop(0, RC)
          def _row(r):
            for cc in range(DP // L):
              plsc.addupdate(sbuf.at[pl.ds(cc * L, L)],
                             rows[r, pl.ds(cc * L, L)])

        pltpu.sync_copy(sbuf, partial_hbm.at[g, t])


def _readout_kernel(partial_ref, ptype_ref, w1a_ref, w1b_ref, b1_ref,
                    w2_ref, b2_ref, w3_ref, b3_ref, out_ref):
  g = jnp.sum(partial_ref[...], axis=1)[:, :D]
  g = jnp.log(g)
  g = jnp.where(jnp.isnan(g), 0.0, g)
  g = jnp.maximum(g, 0.0)
  x = (jnp.dot(g, w1a_ref[...].T, preferred_element_type=jnp.float32)
       + ptype_ref[...] * w1b_ref[...].T + b1_ref[...])
  x = jnp.where(x > 0, x, 0.01 * x)
  x = jnp.dot(x, w2_ref[...].T, preferred_element_type=jnp.float32) + b2_ref[...]
  x = jnp.where(x > 0, x, 0.01 * x)
  x = jnp.dot(x, w3_ref[...].T, preferred_element_type=jnp.float32) + b3_ref[...]
  out_ref[...] = x


def kernel(nodesBatch, backwards_edgeBatch, problemTypeBatch,
           W1, b1, W2, b2, W3, b3):
  nodes_pad = jnp.pad(nodesBatch, ((0, 0), (0, NP - N), (0, DP - D)))
  nodes_pad = nodes_pad.reshape(B * NP, DP)
  dst_l = backwards_edgeBatch[..., 0].reshape(B * EG)
  src_g = (backwards_edgeBatch[..., 1]
           + (jnp.arange(B, dtype=jnp.int32) * NP)[:, None, None]).reshape(B * EG)

  mesh = plsc.VectorSubcoreMesh(core_axis_name="c", subcore_axis_name="s",
                                num_cores=NC, num_subcores=NS)
  mp = pl.kernel(
      _mp_kernel,
      out_type=[jax.ShapeDtypeStruct((B, NS, DP), jnp.float32),
                jax.ShapeDtypeStruct((B * NP, DP), jnp.float32)],
      mesh=mesh,
      compiler_params=pltpu.CompilerParams(use_tc_tiling_on_sc=False),
      scratch_types=[
          pltpu.VMEM_SHARED((NP, DP), jnp.float32),
          pltpu.VMEM((CH,), jnp.int32),
          pltpu.VMEM((CH,), jnp.int32),
          pltpu.VMEM((CH, DP), jnp.float32),
          pltpu.VMEM((DP,), jnp.float32),
          pltpu.SemaphoreType.DMA,
      ],
  )
  partial, _ = mp(nodes_pad, src_g, dst_l)

  out = pl.pallas_call(
      _readout_kernel,
      out_shape=jax.ShapeDtypeStruct((B, 10), jnp.float32),
  )(partial, problemTypeBatch, W1[:, :D], W1[:, D:], b1, W2, b2, W3, b3)
  return out


# speedup vs baseline: 40.1563x; 2.5019x over previous
import functools

import jax
import jax.numpy as jnp
from jax import lax
from jax.experimental import pallas as pl
from jax.experimental.pallas import tpu as pltpu
from jax.experimental.pallas import tpu_sc as plsc

PASSES = 4
NUM_EDGE_SETS = 2
B, N, D, E = 4, 10000, 150, 320000
NP = 10240
DP = 160
EG = NUM_EDGE_SETS * E
NC, NS, L = 2, 16, 16
EPT = EG // NS
CH = 64
NCHUNK = EPT // CH
SB = 25
SCE = SB * CH
NSC = NCHUNK // SB
RPT = NP // NS
RC = 64
NRC = RPT // RC
GPC = B // NC


def _mp_kernel(nodes_hbm, srcg_hbm, dstl_hbm, partial_hbm, nscr_hbm,
               inc_shared, src_blk, dst_blk, rows0, rows1, sbuf,
               gsem0, gsem1):
  c = lax.axis_index("c")
  t = lax.axis_index("s")
  rows = (rows0, rows1)
  gsem = (gsem0, gsem1)

  for gi in range(GPC):
    g = c * GPC + gi
    for p in range(PASSES):
      nsrc = nodes_hbm if p == 0 else nscr_hbm

      lr00 = t * RPT
      pltpu.sync_copy(nsrc.at[pl.ds(g * NP + lr00, RPT)],
                      inc_shared.at[pl.ds(lr00, RPT)])
      plsc.subcore_barrier()

      erow = g * (EG // CH) + t * (EPT // CH)

      @pl.loop(0, NSC)
      def _schunk(s):
        r0 = erow + s * SB
        pltpu.sync_copy(srcg_hbm.at[pl.ds(r0, SB)], src_blk)
        pltpu.sync_copy(dstl_hbm.at[pl.ds(r0, SB)], dst_blk)
        descs = [None, None]
        descs[0] = pltpu.async_copy(nsrc.at[src_blk.at[0]], rows[0], gsem[0])
        for j in range(SB):
          b = j % 2
          if j + 1 < SB:
            descs[1 - b] = pltpu.async_copy(nsrc.at[src_blk.at[j + 1]],
                                            rows[1 - b], gsem[1 - b])
          descs[b].wait()
          pltpu.sync_copy(rows[b], inc_shared.at[dst_blk.at[j]], add=True)

      plsc.subcore_barrier()

      if p < PASSES - 1:
        pltpu.sync_copy(inc_shared.at[pl.ds(lr00, RPT)],
                        nscr_hbm.at[pl.ds(g * NP + lr00, RPT)])
      else:
        for cc in range(DP // L):
          sbuf[pl.ds(cc * L, L)] = jnp.zeros((L,), jnp.float32)
        for k in range(NRC):
          lr0 = t * RPT + k * RC
          pltpu.sync_copy(inc_shared.at[pl.ds(lr0, RC)], rows0)

          @pl.loop(0, RC)
          def _row(r):
            for cc in range(DP // L):
              plsc.addupdate(sbuf.at[pl.ds(cc * L, L)],
                             rows0[r, pl.ds(cc * L, L)])

        pltpu.sync_copy(sbuf, partial_hbm.at[g, t])


def _readout_kernel(partial_ref, ptype_ref, w1a_ref, w1b_ref, b1_ref,
                    w2_ref, b2_ref, w3_ref, b3_ref, out_ref):
  g = jnp.sum(partial_ref[...], axis=1)[:, :D]
  g = jnp.log(g)
  g = jnp.where(jnp.isnan(g), 0.0, g)
  g = jnp.maximum(g, 0.0)
  x = (jnp.dot(g, w1a_ref[...].T, preferred_element_type=jnp.float32)
       + ptype_ref[...] * w1b_ref[...].T + b1_ref[...])
  x = jnp.where(x > 0, x, 0.01 * x)
  x = jnp.dot(x, w2_ref[...].T, preferred_element_type=jnp.float32) + b2_ref[...]
  x = jnp.where(x > 0, x, 0.01 * x)
  x = jnp.dot(x, w3_ref[...].T, preferred_element_type=jnp.float32) + b3_ref[...]
  out_ref[...] = x


def kernel(nodesBatch, backwards_edgeBatch, problemTypeBatch,
           W1, b1, W2, b2, W3, b3):
  nodes_pad = jnp.pad(nodesBatch, ((0, 0), (0, NP - N), (0, DP - D)))
  nodes_pad = nodes_pad.reshape(B * NP, DP)
  dst_l = backwards_edgeBatch[..., 0].reshape(B * EG // CH, CH)
  src_g = (backwards_edgeBatch[..., 1]
           + (jnp.arange(B, dtype=jnp.int32) * NP)[:, None, None]
           ).reshape(B * EG // CH, CH)

  mesh = plsc.VectorSubcoreMesh(core_axis_name="c", subcore_axis_name="s",
                                num_cores=NC, num_subcores=NS)
  mp = pl.kernel(
      _mp_kernel,
      out_type=[jax.ShapeDtypeStruct((B, NS, DP), jnp.float32),
                jax.ShapeDtypeStruct((B * NP, DP), jnp.float32)],
      mesh=mesh,
      compiler_params=pltpu.CompilerParams(use_tc_tiling_on_sc=False),
      scratch_types=[
          pltpu.VMEM_SHARED((NP, DP), jnp.float32),
          pltpu.VMEM((SB, CH), jnp.int32),
          pltpu.VMEM((SB, CH), jnp.int32),
          pltpu.VMEM((CH, DP), jnp.float32),
          pltpu.VMEM((CH, DP), jnp.float32),
          pltpu.VMEM((DP,), jnp.float32),
          pltpu.SemaphoreType.DMA,
          pltpu.SemaphoreType.DMA,
      ],
  )
  partial, _ = mp(nodes_pad, src_g, dst_l)

  out = pl.pallas_call(
      _readout_kernel,
      out_shape=jax.ShapeDtypeStruct((B, 10), jnp.float32),
  )(partial, problemTypeBatch, W1[:, :D], W1[:, D:], b1, W2, b2, W3, b3)
  return out


# speedup vs baseline: 59.2149x; 1.4746x over previous
import functools

import jax
import jax.numpy as jnp
from jax import lax
from jax.experimental import pallas as pl
from jax.experimental.pallas import tpu as pltpu
from jax.experimental.pallas import tpu_sc as plsc

PASSES = 4
NUM_EDGE_SETS = 2
B, N, D, E = 4, 10000, 150, 320000
NP = 10240
DP = 160
EG = NUM_EDGE_SETS * E
NC, NS, L = 2, 16, 16
EPT = EG // NS
CH = 80
NCHUNK = EPT // CH
SB = 25
NSC = NCHUNK // SB
RPT = NP // NS
GPC = B // NC


def _mp_kernel(nodes_hbm, srcg_hbm, dstl_hbm, nscr_hbm,
               inc_shared, src_blk, dst_blk, rows0, rows1, gsem0, gsem1):
  c = lax.axis_index("c")
  t = lax.axis_index("s")
  rows = (rows0, rows1)
  gsem = (gsem0, gsem1)

  for gi in range(GPC):
    g = c * GPC + gi
    for p in range(PASSES):
      nsrc = nodes_hbm if p == 0 else nscr_hbm

      lr00 = t * RPT
      pltpu.sync_copy(nsrc.at[pl.ds(g * NP + lr00, RPT)],
                      inc_shared.at[pl.ds(lr00, RPT)])
      plsc.subcore_barrier()

      erow = g * (EG // CH) + t * (EPT // CH)

      @pl.loop(0, NSC)
      def _schunk(s):
        r0 = erow + s * SB
        pltpu.sync_copy(srcg_hbm.at[pl.ds(r0, SB)], src_blk)
        pltpu.sync_copy(dstl_hbm.at[pl.ds(r0, SB)], dst_blk)
        descs = [None, None]
        descs[0] = pltpu.async_copy(nsrc.at[src_blk.at[0]], rows[0], gsem[0])
        for j in range(SB):
          b = j % 2
          if j + 1 < SB:
            descs[1 - b] = pltpu.async_copy(nsrc.at[src_blk.at[j + 1]],
                                            rows[1 - b], gsem[1 - b])
          descs[b].wait()
          pltpu.sync_copy(rows[b], inc_shared.at[dst_blk.at[j]], add=True)

      plsc.subcore_barrier()

      pltpu.sync_copy(inc_shared.at[pl.ds(lr00, RPT)],
                      nscr_hbm.at[pl.ds(g * NP + lr00, RPT)])


def _readout_kernel(nfin_ref, ptype_ref, w1a_ref, w1b_ref, b1_ref,
                    w2_ref, b2_ref, w3_ref, b3_ref, out_ref):
  g = jnp.sum(nfin_ref[...].astype(jnp.float32), axis=1)[:, :D]
  g = jnp.log(g)
  g = jnp.where(jnp.isnan(g), 0.0, g)
  g = jnp.maximum(g, 0.0)
  x = (jnp.dot(g, w1a_ref[...].T, preferred_element_type=jnp.float32)
       + ptype_ref[...] * w1b_ref[...].T + b1_ref[...])
  x = jnp.where(x > 0, x, 0.01 * x)
  x = jnp.dot(x, w2_ref[...].T, preferred_element_type=jnp.float32) + b2_ref[...]
  x = jnp.where(x > 0, x, 0.01 * x)
  x = jnp.dot(x, w3_ref[...].T, preferred_element_type=jnp.float32) + b3_ref[...]
  out_ref[...] = x


def kernel(nodesBatch, backwards_edgeBatch, problemTypeBatch,
           W1, b1, W2, b2, W3, b3):
  nodes_pad = jnp.pad(nodesBatch, ((0, 0), (0, NP - N), (0, DP - D)))
  nodes_pad = nodes_pad.reshape(B * NP, DP).astype(jnp.bfloat16)
  dst_l = backwards_edgeBatch[..., 0].reshape(B * EG // CH, CH)
  src_g = (backwards_edgeBatch[..., 1]
           + (jnp.arange(B, dtype=jnp.int32) * NP)[:, None, None]
           ).reshape(B * EG // CH, CH)

  mesh = plsc.VectorSubcoreMesh(core_axis_name="c", subcore_axis_name="s",
                                num_cores=NC, num_subcores=NS)
  mp = pl.kernel(
      _mp_kernel,
      out_type=jax.ShapeDtypeStruct((B * NP, DP), jnp.bfloat16),
      mesh=mesh,
      compiler_params=pltpu.CompilerParams(use_tc_tiling_on_sc=False),
      scratch_types=[
          pltpu.VMEM_SHARED((NP, DP), jnp.bfloat16),
          pltpu.VMEM((SB, CH), jnp.int32),
          pltpu.VMEM((SB, CH), jnp.int32),
          pltpu.VMEM((CH, DP), jnp.bfloat16),
          pltpu.VMEM((CH, DP), jnp.bfloat16),
          pltpu.SemaphoreType.DMA,
          pltpu.SemaphoreType.DMA,
      ],
  )
  nfin = mp(nodes_pad, src_g, dst_l).reshape(B, NP, DP)

  out = pl.pallas_call(
      _readout_kernel,
      out_shape=jax.ShapeDtypeStruct((B, 10), jnp.float32),
  )(nfin, problemTypeBatch, W1[:, :D], W1[:, D:], b1, W2, b2, W3, b3)
  return out


# speedup vs baseline: 70.0878x; 1.1836x over previous
import functools

import jax
import jax.numpy as jnp
from jax import lax
from jax.experimental import pallas as pl
from jax.experimental.pallas import tpu as pltpu
from jax.experimental.pallas import tpu_sc as plsc

PASSES = 4
NUM_EDGE_SETS = 2
B, N, D, E = 4, 10000, 150, 320000
NP = 10240
DP = 160
EG = NUM_EDGE_SETS * E
NC, NS, L = 2, 16, 16
EPT = EG // NS
CH = 80
NCHUNK = EPT // CH
SB = 25
NSC = NCHUNK // SB
RPT = NP // NS
GPC = B // NC


def _mp_kernel(nodes_hbm, srcg_hbm, dstl_hbm, nscr_hbm,
               inc_shared, src_blk, dst_blk, rows0, rows1, rows2,
               gsem0, gsem1, gsem2, ssem0, ssem1, ssem2):
  c = lax.axis_index("c")
  t = lax.axis_index("s")
  rows = (rows0, rows1, rows2)
  gsem = (gsem0, gsem1, gsem2)
  ssem = (ssem0, ssem1, ssem2)

  for gi in range(GPC):
    g = c * GPC + gi
    for p in range(PASSES):
      nsrc = nodes_hbm if p == 0 else nscr_hbm

      lr00 = t * RPT
      pltpu.sync_copy(nsrc.at[pl.ds(g * NP + lr00, RPT)],
                      inc_shared.at[pl.ds(lr00, RPT)])
      plsc.subcore_barrier()

      erow = g * (EG // CH) + t * (EPT // CH)

      @pl.loop(0, NSC)
      def _schunk(s):
        r0 = erow + s * SB
        pltpu.sync_copy(srcg_hbm.at[pl.ds(r0, SB)], src_blk)
        pltpu.sync_copy(dstl_hbm.at[pl.ds(r0, SB)], dst_blk)
        gd = [None] * SB
        sd = [None] * SB
        gd[0] = pltpu.async_copy(nsrc.at[src_blk.at[0]], rows[0], gsem[0])
        gd[1] = pltpu.async_copy(nsrc.at[src_blk.at[1]], rows[1], gsem[1])
        for j in range(SB):
          b = j % 3
          gd[j].wait()
          sd[j] = pltpu.async_copy(rows[b], inc_shared.at[dst_blk.at[j]],
                                   ssem[b], add=True)
          if j + 2 < SB:
            if j >= 1:
              sd[j - 1].wait()
            gd[j + 2] = pltpu.async_copy(nsrc.at[src_blk.at[j + 2]],
                                         rows[(j + 2) % 3], gsem[(j + 2) % 3])
        sd[SB - 3].wait()
        sd[SB - 2].wait()
        sd[SB - 1].wait()

      plsc.subcore_barrier()

      pltpu.sync_copy(inc_shared.at[pl.ds(lr00, RPT)],
                      nscr_hbm.at[pl.ds(g * NP + lr00, RPT)])


def _readout_kernel(nfin_ref, ptype_ref, w1a_ref, w1b_ref, b1_ref,
                    w2_ref, b2_ref, w3_ref, b3_ref, out_ref):
  g = jnp.sum(nfin_ref[...].astype(jnp.float32), axis=1)[:, :D]
  g = jnp.log(g)
  g = jnp.where(jnp.isnan(g), 0.0, g)
  g = jnp.maximum(g, 0.0)
  x = (jnp.dot(g, w1a_ref[...].T, preferred_element_type=jnp.float32)
       + ptype_ref[...] * w1b_ref[...].T + b1_ref[...])
  x = jnp.where(x > 0, x, 0.01 * x)
  x = jnp.dot(x, w2_ref[...].T, preferred_element_type=jnp.float32) + b2_ref[...]
  x = jnp.where(x > 0, x, 0.01 * x)
  x = jnp.dot(x, w3_ref[...].T, preferred_element_type=jnp.float32) + b3_ref[...]
  out_ref[...] = x


def kernel(nodesBatch, backwards_edgeBatch, problemTypeBatch,
           W1, b1, W2, b2, W3, b3):
  nodes_pad = jnp.pad(nodesBatch, ((0, 0), (0, NP - N), (0, DP - D)))
  nodes_pad = nodes_pad.reshape(B * NP, DP).astype(jnp.bfloat16)
  dst_l = backwards_edgeBatch[..., 0].reshape(B * EG // CH, CH)
  src_g = (backwards_edgeBatch[..., 1]
           + (jnp.arange(B, dtype=jnp.int32) * NP)[:, None, None]
           ).reshape(B * EG // CH, CH)

  mesh = plsc.VectorSubcoreMesh(core_axis_name="c", subcore_axis_name="s",
                                num_cores=NC, num_subcores=NS)
  mp = pl.kernel(
      _mp_kernel,
      out_type=jax.ShapeDtypeStruct((B * NP, DP), jnp.bfloat16),
      mesh=mesh,
      compiler_params=pltpu.CompilerParams(use_tc_tiling_on_sc=False),
      scratch_types=[
          pltpu.VMEM_SHARED((NP, DP), jnp.bfloat16),
          pltpu.VMEM((SB, CH), jnp.int32),
          pltpu.VMEM((SB, CH), jnp.int32),
          pltpu.VMEM((CH, DP), jnp.bfloat16),
          pltpu.VMEM((CH, DP), jnp.bfloat16),
          pltpu.VMEM((CH, DP), jnp.bfloat16),
          pltpu.SemaphoreType.DMA,
          pltpu.SemaphoreType.DMA,
          pltpu.SemaphoreType.DMA,
          pltpu.SemaphoreType.DMA,
          pltpu.SemaphoreType.DMA,
          pltpu.SemaphoreType.DMA,
      ],
  )
  nfin = mp(nodes_pad, src_g, dst_l).reshape(B, NP, DP)

  out = pl.pallas_call(
      _readout_kernel,
      out_shape=jax.ShapeDtypeStruct((B, 10), jnp.float32),
  )(nfin, problemTypeBatch, W1[:, :D], W1[:, D:], b1, W2, b2, W3, b3)
  return out
